# baseline (speedup 1.0000x reference)
import functools

import jax
import jax.numpy as jnp
import numpy as np
from jax import lax
from jax.experimental import pallas as pl
from jax.experimental.pallas import tpu as pltpu
from jax.experimental.pallas import tpu_sc as plsc

N_NODES = 10000
N_EDGES = 320000
NODE_DIM = 128
EDGE_DIM = 16
EDGE_DIM_OUT = 128
PKD = EDGE_DIM_OUT // 2

NW = 32
E_BLK = 64
NB = N_EDGES // E_BLK
NB_MAIN = 156
NB_EXTRA = NB - NW * NB_MAIN
PF = NB_MAIN + 1

_PERM = np.concatenate(
    [np.concatenate([np.arange(0, 32, 2), np.arange(1, 32, 2)]) + 32 * g
     for g in range(EDGE_DIM_OUT // 32)])


def _node_proj_body(x_ref, ws_ref, wd_ref, ps_ref, pd_ref):
    x = x_ref[...]
    ps_ref[...] = jnp.dot(x, ws_ref[...], preferred_element_type=jnp.float32)
    pd_ref[...] = jnp.dot(x, wd_ref[...], preferred_element_type=jnp.float32)


def _node_proj(node_emb, ws_t, wd_t):
    blk = 2000
    grid = (N_NODES // blk,)
    return pl.pallas_call(
        _node_proj_body,
        grid=grid,
        in_specs=[
            pl.BlockSpec((blk, NODE_DIM), lambda i: (i, 0)),
            pl.BlockSpec((NODE_DIM, NODE_DIM), lambda i: (0, 0)),
            pl.BlockSpec((NODE_DIM, NODE_DIM), lambda i: (0, 0)),
        ],
        out_specs=[
            pl.BlockSpec((blk, EDGE_DIM_OUT), lambda i: (i, 0)),
            pl.BlockSpec((blk, EDGE_DIM_OUT), lambda i: (i, 0)),
        ],
        out_shape=[
            jax.ShapeDtypeStruct((N_NODES, EDGE_DIM_OUT), jnp.float32),
            jax.ShapeDtypeStruct((N_NODES, EDGE_DIM_OUT), jnp.float32),
        ],
    )(node_emb, ws_t, wd_t)


_sc_mesh = plsc.VectorSubcoreMesh(core_axis_name="c", subcore_axis_name="s")


@functools.partial(
    pl.kernel,
    out_type=jax.ShapeDtypeStruct((NB * E_BLK * PKD,), jnp.int32),
    mesh=_sc_mesh,
    scratch_types=[
        pltpu.VMEM((PF * E_BLK,), jnp.int32),
        pltpu.VMEM((PF * E_BLK,), jnp.int32),
        pltpu.VMEM((E_BLK, EDGE_DIM_OUT), jnp.float32),
        pltpu.VMEM((E_BLK, EDGE_DIM_OUT), jnp.float32),
        pltpu.VMEM((E_BLK, EDGE_DIM_OUT), jnp.float32),
        pltpu.VMEM((E_BLK, EDGE_DIM_OUT), jnp.float32),
        pltpu.VMEM((E_BLK, EDGE_DIM_OUT), jnp.float32),
        pltpu.VMEM((E_BLK, EDGE_DIM_OUT), jnp.float32),
        pltpu.VMEM((E_BLK * PKD,), jnp.int32),
        pltpu.VMEM((E_BLK * PKD,), jnp.int32),
        pltpu.VMEM((E_BLK * PKD,), jnp.int32),
        pltpu.SemaphoreType.DMA,
        pltpu.SemaphoreType.DMA,
        pltpu.SemaphoreType.DMA,
        pltpu.SemaphoreType.DMA,
        pltpu.SemaphoreType.DMA,
        pltpu.SemaphoreType.DMA,
    ],
)
def _sc_gather_add(ps_hbm, pd_hbm, src_hbm, dst_hbm, out_hbm,
                   idx_s, idx_d, rs0, rs1, rs2, rd0, rd1, rd2,
                   go0, go1, go2, sg0, sg1, sg2, so0, so1, so2):
    num_c = lax.axis_size("c")
    wid = lax.axis_index("s") * num_c + lax.axis_index("c")
    start = wid * NB_MAIN + jnp.minimum(wid, NB_EXTRA)
    pf_start = jnp.minimum(start, NB - PF)
    off = start - pf_start

    pltpu.sync_copy(src_hbm.at[pl.ds(pf_start * E_BLK, PF * E_BLK)], idx_s)
    pltpu.sync_copy(dst_hbm.at[pl.ds(pf_start * E_BLK, PF * E_BLK)], idx_d)

    RS = (rs0, rs1, rs2)
    RD = (rd0, rd1, rd2)
    GO = (go0, go1, go2)
    SG = (sg0, sg1, sg2)
    SO = (so0, so1, so2)

    def issue_gather(p, loc):
        k = (loc + off) * E_BLK
        pltpu.async_copy(ps_hbm.at[idx_s.at[pl.ds(k, E_BLK)]], RS[p], SG[p])
        pltpu.async_copy(pd_hbm.at[idx_d.at[pl.ds(k, E_BLK)]], RD[p], SG[p])

    def wait_gather(p, loc):
        k = (loc + off) * E_BLK
        pltpu.make_async_copy(ps_hbm.at[idx_s.at[pl.ds(k, E_BLK)]], RS[p], SG[p]).wait()
        pltpu.make_async_copy(pd_hbm.at[idx_d.at[pl.ds(k, E_BLK)]], RD[p], SG[p]).wait()

    BSZ = E_BLK * PKD

    def issue_out(p, loc):
        pltpu.async_copy(GO[p], out_hbm.at[pl.ds((start + loc) * BSZ, BSZ)], SO[p])

    def wait_out(p):
        pltpu.make_async_copy(GO[p], out_hbm.at[pl.ds(start * BSZ, BSZ)], SO[p]).wait()

    hi_mask = jnp.full((16,), -65536, dtype=jnp.int32)
    half = jnp.full((16,), 32768, dtype=jnp.int32)

    def compute(p):
        rs, rd, go = RS[p], RD[p], GO[p]

        @plsc.parallel_loop(0, E_BLK, unroll=2)
        def _rows(r):
            for cc in range(EDGE_DIM_OUT // 32):
                sl_a = pl.ds(cc * 32, 16)
                sl_b = pl.ds(cc * 32 + 16, 16)
                a = rs[r, sl_a] + rd[r, sl_a]
                bvals = rs[r, sl_b] + rd[r, sl_b]
                a_r = lax.bitcast_convert_type(a, jnp.int32) + half
                b_r = lax.bitcast_convert_type(bvals, jnp.int32) + half
                word = lax.shift_right_logical(a_r, 16) | (b_r & hi_mask)
                go[pl.ds(r * PKD + cc * 16, 16)] = word

    issue_gather(0, 0)

    def iter_body(i3, carry):
        for j in range(3):
            loc = 3 * i3 + j
            q = (j + 1) % 3
            nxt = loc + 1

            @pl.when(nxt < NB_MAIN)
            def _():
                issue_gather(q, nxt)

            wait_gather(j, loc)

            @pl.when(loc >= 3)
            def _():
                wait_out(j)

            compute(j)
            issue_out(j, loc)
        return carry

    lax.fori_loop(0, NB_MAIN // 3, iter_body, 0)
    wait_out(0)
    wait_out(1)
    wait_out(2)

    @pl.when(wid < NB_EXTRA)
    def _():
        issue_gather(0, NB_MAIN)
        wait_gather(0, NB_MAIN)
        compute(0)
        issue_out(0, NB_MAIN)
        wait_out(0)


def _finish_body(ee_ref, g_ref, we_ref, b_ref, o_ref):
    t = jnp.dot(ee_ref[...], we_ref[...], preferred_element_type=jnp.float32)
    g = g_ref[...].astype(jnp.float32)
    o_ref[...] = jnp.maximum(t + g + b_ref[...], 0.0)


def _finish(edge_emb, g, we_t, b2d):
    blk = 8000
    grid = (N_EDGES // blk,)
    return pl.pallas_call(
        _finish_body,
        grid=grid,
        in_specs=[
            pl.BlockSpec((blk, EDGE_DIM), lambda i: (i, 0)),
            pl.BlockSpec((blk, EDGE_DIM_OUT), lambda i: (i, 0)),
            pl.BlockSpec((EDGE_DIM, EDGE_DIM_OUT), lambda i: (0, 0)),
            pl.BlockSpec((1, EDGE_DIM_OUT), lambda i: (0, 0)),
        ],
        out_specs=pl.BlockSpec((blk, EDGE_DIM_OUT), lambda i: (i, 0)),
        out_shape=jax.ShapeDtypeStruct((N_EDGES, EDGE_DIM_OUT), jnp.float32),
    )(edge_emb, g, we_t, b2d)


def kernel(edge_index, edge_emb, node_emb, W, b):
    ei = edge_index.astype(jnp.int32)
    src1 = ei[0]
    dst1 = ei[1]

    we_t = W[:, :EDGE_DIM].T
    ws_t = W[:, EDGE_DIM:EDGE_DIM + NODE_DIM].T
    wd_t = W[:, EDGE_DIM + NODE_DIM:].T
    ws_t = ws_t[:, _PERM]
    wd_t = wd_t[:, _PERM]

    ps, pd = _node_proj(node_emb, ws_t, wd_t)
    g = _sc_gather_add(ps, pd, src1, dst1)
    g2 = jax.lax.bitcast_convert_type(
        g.reshape(N_EDGES, PKD), jnp.bfloat16).reshape(N_EDGES, EDGE_DIM_OUT)
    return _finish(edge_emb, g2, we_t, b.reshape(1, EDGE_DIM_OUT))


# speedup vs baseline: 2.7627x; 2.7627x over previous
import functools

import jax
import jax.numpy as jnp
from jax import lax
from jax.experimental import pallas as pl
from jax.experimental.pallas import tpu as pltpu
from jax.experimental.pallas import tpu_sc as plsc

N_NODES = 10000
N_EDGES = 320000
NODE_DIM = 128
EDGE_DIM = 16
EDGE_DIM_OUT = 128
PKD = EDGE_DIM_OUT // 2

NW = 32
E_BLK = 64
NB = N_EDGES // E_BLK
NB_MAIN = 156
NB_EXTRA = NB - NW * NB_MAIN
PF = NB_MAIN + 1

def _node_proj_body(x_ref, ws_ref, wd_ref, ps_ref, pd_ref):
    x = x_ref[...]
    ps_ref[...] = jnp.dot(x, ws_ref[...], preferred_element_type=jnp.float32)
    pd_ref[...] = jnp.dot(x, wd_ref[...], preferred_element_type=jnp.float32)


def _node_proj(node_emb, ws_t, wd_t):
    blk = 2000
    grid = (N_NODES // blk,)
    return pl.pallas_call(
        _node_proj_body,
        grid=grid,
        in_specs=[
            pl.BlockSpec((blk, NODE_DIM), lambda i: (i, 0)),
            pl.BlockSpec((NODE_DIM, NODE_DIM), lambda i: (0, 0)),
            pl.BlockSpec((NODE_DIM, NODE_DIM), lambda i: (0, 0)),
        ],
        out_specs=[
            pl.BlockSpec((blk, EDGE_DIM_OUT), lambda i: (i, 0)),
            pl.BlockSpec((blk, EDGE_DIM_OUT), lambda i: (i, 0)),
        ],
        out_shape=[
            jax.ShapeDtypeStruct((N_NODES, EDGE_DIM_OUT), jnp.float32),
            jax.ShapeDtypeStruct((N_NODES, EDGE_DIM_OUT), jnp.float32),
        ],
    )(node_emb, ws_t, wd_t)


_sc_mesh = plsc.VectorSubcoreMesh(core_axis_name="c", subcore_axis_name="s")


@functools.partial(
    pl.kernel,
    out_type=jax.ShapeDtypeStruct((N_EDGES // 2, EDGE_DIM_OUT), jnp.int32),
    mesh=_sc_mesh,
    scratch_types=[
        pltpu.VMEM((PF * E_BLK,), jnp.int32),
        pltpu.VMEM((PF * E_BLK,), jnp.int32),
        pltpu.VMEM((E_BLK, EDGE_DIM_OUT), jnp.float32),
        pltpu.VMEM((E_BLK, EDGE_DIM_OUT), jnp.float32),
        pltpu.VMEM((E_BLK, EDGE_DIM_OUT), jnp.float32),
        pltpu.VMEM((E_BLK, EDGE_DIM_OUT), jnp.float32),
        pltpu.VMEM((E_BLK, EDGE_DIM_OUT), jnp.float32),
        pltpu.VMEM((E_BLK, EDGE_DIM_OUT), jnp.float32),
        pltpu.VMEM((E_BLK // 2, EDGE_DIM_OUT), jnp.int32),
        pltpu.VMEM((E_BLK // 2, EDGE_DIM_OUT), jnp.int32),
        pltpu.VMEM((E_BLK // 2, EDGE_DIM_OUT), jnp.int32),
        pltpu.SemaphoreType.DMA,
        pltpu.SemaphoreType.DMA,
        pltpu.SemaphoreType.DMA,
        pltpu.SemaphoreType.DMA,
        pltpu.SemaphoreType.DMA,
        pltpu.SemaphoreType.DMA,
    ],
)
def _sc_gather_add(ps_hbm, pd_hbm, src_hbm, dst_hbm, out_hbm,
                   idx_s, idx_d, rs0, rs1, rs2, rd0, rd1, rd2,
                   go0, go1, go2, sg0, sg1, sg2, so0, so1, so2):
    num_c = lax.axis_size("c")
    wid = lax.axis_index("s") * num_c + lax.axis_index("c")
    start = wid * NB_MAIN + jnp.minimum(wid, NB_EXTRA)
    pf_start = jnp.minimum(start, NB - PF)
    off = start - pf_start

    pltpu.sync_copy(src_hbm.at[pl.ds(pf_start * E_BLK, PF * E_BLK)], idx_s)
    pltpu.sync_copy(dst_hbm.at[pl.ds(pf_start * E_BLK, PF * E_BLK)], idx_d)

    RS = (rs0, rs1, rs2)
    RD = (rd0, rd1, rd2)
    GO = (go0, go1, go2)
    SG = (sg0, sg1, sg2)
    SO = (so0, so1, so2)

    def issue_gather(p, loc):
        k = (loc + off) * E_BLK
        pltpu.async_copy(ps_hbm.at[idx_s.at[pl.ds(k, E_BLK)]], RS[p], SG[p])
        pltpu.async_copy(pd_hbm.at[idx_d.at[pl.ds(k, E_BLK)]], RD[p], SG[p])

    def wait_gather(p, loc):
        k = (loc + off) * E_BLK
        pltpu.make_async_copy(ps_hbm.at[idx_s.at[pl.ds(k, E_BLK)]], RS[p], SG[p]).wait()
        pltpu.make_async_copy(pd_hbm.at[idx_d.at[pl.ds(k, E_BLK)]], RD[p], SG[p]).wait()

    HB = E_BLK // 2

    def issue_out(p, loc):
        pltpu.async_copy(GO[p], out_hbm.at[pl.ds((start + loc) * HB, HB)], SO[p])

    def wait_out(p):
        pltpu.make_async_copy(GO[p], out_hbm.at[pl.ds(start * HB, HB)], SO[p]).wait()

    hi_mask = jnp.full((16,), -65536, dtype=jnp.int32)
    half = jnp.full((16,), 32768, dtype=jnp.int32)

    def compute(p):
        rs, rd, go = RS[p], RD[p], GO[p]

        @plsc.parallel_loop(0, E_BLK // 2, unroll=2)
        def _pairs(m):
            for cc in range(EDGE_DIM_OUT // 16):
                sl = pl.ds(cc * 16, 16)
                a = rs[2 * m, sl] + rd[2 * m, sl]
                bvals = rs[2 * m + 1, sl] + rd[2 * m + 1, sl]
                a_r = lax.bitcast_convert_type(a, jnp.int32) + half
                b_r = lax.bitcast_convert_type(bvals, jnp.int32) + half
                go[m, sl] = lax.shift_right_logical(a_r, 16) | (b_r & hi_mask)

    issue_gather(0, 0)

    def iter_body(i3, carry):
        for j in range(3):
            loc = 3 * i3 + j
            q = (j + 1) % 3
            nxt = loc + 1

            @pl.when(nxt < NB_MAIN)
            def _():
                issue_gather(q, nxt)

            wait_gather(j, loc)

            @pl.when(loc >= 3)
            def _():
                wait_out(j)

            compute(j)
            issue_out(j, loc)
        return carry

    lax.fori_loop(0, NB_MAIN // 3, iter_body, 0)
    wait_out(0)
    wait_out(1)
    wait_out(2)

    @pl.when(wid < NB_EXTRA)
    def _():
        issue_gather(0, NB_MAIN)
        wait_gather(0, NB_MAIN)
        compute(0)
        issue_out(0, NB_MAIN)
        wait_out(0)


def _finish_body(ee_ref, g_ref, we_ref, b_ref, o_ref):
    t = jnp.dot(ee_ref[...], we_ref[...], preferred_element_type=jnp.float32)
    w = g_ref[...]
    lo = lax.bitcast_convert_type(w << 16, jnp.float32)
    hi = lax.bitcast_convert_type(w & -65536, jnp.float32)
    g = jnp.stack([lo, hi], axis=1).reshape(t.shape)
    o_ref[...] = jnp.maximum(t + g + b_ref[...], 0.0)


def _finish(edge_emb, g, we_t, b2d):
    blk = 8000
    grid = (N_EDGES // blk,)
    return pl.pallas_call(
        _finish_body,
        grid=grid,
        in_specs=[
            pl.BlockSpec((blk, EDGE_DIM), lambda i: (i, 0)),
            pl.BlockSpec((blk // 2, EDGE_DIM_OUT), lambda i: (i, 0)),
            pl.BlockSpec((EDGE_DIM, EDGE_DIM_OUT), lambda i: (0, 0)),
            pl.BlockSpec((1, EDGE_DIM_OUT), lambda i: (0, 0)),
        ],
        out_specs=pl.BlockSpec((blk, EDGE_DIM_OUT), lambda i: (i, 0)),
        out_shape=jax.ShapeDtypeStruct((N_EDGES, EDGE_DIM_OUT), jnp.float32),
    )(edge_emb, g, we_t, b2d)


def kernel(edge_index, edge_emb, node_emb, W, b):
    ei = edge_index.astype(jnp.int32)
    src1 = ei[0]
    dst1 = ei[1]

    we_t = W[:, :EDGE_DIM].T
    ws_t = W[:, EDGE_DIM:EDGE_DIM + NODE_DIM].T
    wd_t = W[:, EDGE_DIM + NODE_DIM:].T

    ps, pd = _node_proj(node_emb, ws_t, wd_t)
    g = _sc_gather_add(ps, pd, src1, dst1)
    return _finish(edge_emb, g, we_t, b.reshape(1, EDGE_DIM_OUT))


# speedup vs baseline: 2.8316x; 1.0249x over previous
import functools

import jax
import jax.numpy as jnp
from jax import lax
from jax.experimental import pallas as pl
from jax.experimental.pallas import tpu as pltpu
from jax.experimental.pallas import tpu_sc as plsc

N_NODES = 10000
N_EDGES = 320000
NODE_DIM = 128
EDGE_DIM = 16
EDGE_DIM_OUT = 128

NW = 32
E_BLK = 128
NB = N_EDGES // E_BLK
NB_MAIN = 78
NB_EXTRA = NB - NW * NB_MAIN
PF = NB_MAIN + 1


def _node_proj_body(x_ref, ws_ref, wd_ref, ps_ref, pd_ref):
    x = x_ref[...]
    ps_ref[...] = jnp.dot(x, ws_ref[...], preferred_element_type=jnp.float32)
    pd_ref[...] = jnp.dot(x, wd_ref[...], preferred_element_type=jnp.float32)


def _node_proj(node_emb, ws_t, wd_t):
    blk = 2000
    grid = (N_NODES // blk,)
    return pl.pallas_call(
        _node_proj_body,
        grid=grid,
        in_specs=[
            pl.BlockSpec((blk, NODE_DIM), lambda i: (i, 0)),
            pl.BlockSpec((NODE_DIM, NODE_DIM), lambda i: (0, 0)),
            pl.BlockSpec((NODE_DIM, NODE_DIM), lambda i: (0, 0)),
        ],
        out_specs=[
            pl.BlockSpec((blk, EDGE_DIM_OUT), lambda i: (i, 0)),
            pl.BlockSpec((blk, EDGE_DIM_OUT), lambda i: (i, 0)),
        ],
        out_shape=[
            jax.ShapeDtypeStruct((N_NODES, EDGE_DIM_OUT), jnp.float32),
            jax.ShapeDtypeStruct((N_NODES, EDGE_DIM_OUT), jnp.float32),
        ],
    )(node_emb, ws_t, wd_t)


_sc_mesh = plsc.VectorSubcoreMesh(core_axis_name="c", subcore_axis_name="s")


@functools.partial(
    pl.kernel,
    out_type=jax.ShapeDtypeStruct((NB, E_BLK, EDGE_DIM_OUT), jnp.float32),
    mesh=_sc_mesh,
    scratch_types=[
        pltpu.VMEM((PF * E_BLK,), jnp.int32),
        pltpu.VMEM((PF * E_BLK,), jnp.int32),
        pltpu.VMEM((E_BLK, EDGE_DIM_OUT), jnp.float32),
        pltpu.VMEM((E_BLK, EDGE_DIM_OUT), jnp.float32),
        pltpu.VMEM((E_BLK, EDGE_DIM_OUT), jnp.float32),
        pltpu.VMEM((E_BLK, EDGE_DIM_OUT), jnp.float32),
        pltpu.VMEM((E_BLK, EDGE_DIM_OUT), jnp.float32),
        pltpu.VMEM((E_BLK, EDGE_DIM_OUT), jnp.float32),
        pltpu.SemaphoreType.DMA,
        pltpu.SemaphoreType.DMA,
        pltpu.SemaphoreType.DMA,
        pltpu.SemaphoreType.DMA,
        pltpu.SemaphoreType.DMA,
        pltpu.SemaphoreType.DMA,
    ],
)
def _sc_gather_add(ps_hbm, pd_hbm, src_hbm, dst_hbm, out_hbm,
                   idx_s, idx_d, rs0, rs1, rs2, rd0, rd1, rd2,
                   sg0, sg1, sg2, so0, so1, so2):
    num_c = lax.axis_size("c")
    wid = lax.axis_index("s") * num_c + lax.axis_index("c")
    start = wid * NB_MAIN + jnp.minimum(wid, NB_EXTRA)
    pf_start = jnp.minimum(start, NB - PF)
    off = start - pf_start

    pltpu.sync_copy(src_hbm.at[pl.ds(pf_start * E_BLK, PF * E_BLK)], idx_s)
    pltpu.sync_copy(dst_hbm.at[pl.ds(pf_start * E_BLK, PF * E_BLK)], idx_d)

    RS = (rs0, rs1, rs2)
    RD = (rd0, rd1, rd2)
    SG = (sg0, sg1, sg2)
    SO = (so0, so1, so2)

    def issue_gather(p, loc):
        k = (loc + off) * E_BLK
        pltpu.async_copy(ps_hbm.at[idx_s.at[pl.ds(k, E_BLK)]], RS[p], SG[p])
        pltpu.async_copy(pd_hbm.at[idx_d.at[pl.ds(k, E_BLK)]], RD[p], SG[p])

    def wait_gather(p, loc):
        k = (loc + off) * E_BLK
        pltpu.make_async_copy(ps_hbm.at[idx_s.at[pl.ds(k, E_BLK)]], RS[p], SG[p]).wait()
        pltpu.make_async_copy(pd_hbm.at[idx_d.at[pl.ds(k, E_BLK)]], RD[p], SG[p]).wait()

    def issue_out(p, loc):
        pltpu.async_copy(RS[p], out_hbm.at[start + loc], SO[p])

    def wait_out(p):
        pltpu.make_async_copy(RS[p], out_hbm.at[start], SO[p]).wait()

    def compute(p):
        rs, rd = RS[p], RD[p]

        @plsc.parallel_loop(0, E_BLK, unroll=2)
        def _rows(r):
            for cc in range(EDGE_DIM_OUT // 16):
                sl = pl.ds(cc * 16, 16)
                rs[r, sl] = rs[r, sl] + rd[r, sl]

    issue_gather(0, 0)

    def iter_body(i3, carry):
        for j in range(3):
            loc = 3 * i3 + j
            q = (j + 1) % 3
            nxt = loc + 1

            @pl.when(nxt < NB_MAIN)
            def _():
                @pl.when(nxt >= 3)
                def _():
                    wait_out(q)
                issue_gather(q, nxt)

            wait_gather(j, loc)
            compute(j)
            issue_out(j, loc)
        return carry

    lax.fori_loop(0, NB_MAIN // 3, iter_body, 0)
    wait_out(0)
    wait_out(1)
    wait_out(2)

    @pl.when(wid < NB_EXTRA)
    def _():
        issue_gather(0, NB_MAIN)
        wait_gather(0, NB_MAIN)
        compute(0)
        issue_out(0, NB_MAIN)
        wait_out(0)


def _finish_body(ee_ref, g_ref, we_ref, b_ref, o_ref):
    t = jnp.dot(ee_ref[...], we_ref[...], preferred_element_type=jnp.float32)
    o_ref[...] = jnp.maximum(t + g_ref[...] + b_ref[...], 0.0)


def _finish(edge_emb, g, we_t, b2d):
    blk = 8000
    grid = (N_EDGES // blk,)
    return pl.pallas_call(
        _finish_body,
        grid=grid,
        in_specs=[
            pl.BlockSpec((blk, EDGE_DIM), lambda i: (i, 0)),
            pl.BlockSpec((blk, EDGE_DIM_OUT), lambda i: (i, 0)),
            pl.BlockSpec((EDGE_DIM, EDGE_DIM_OUT), lambda i: (0, 0)),
            pl.BlockSpec((1, EDGE_DIM_OUT), lambda i: (0, 0)),
        ],
        out_specs=pl.BlockSpec((blk, EDGE_DIM_OUT), lambda i: (i, 0)),
        out_shape=jax.ShapeDtypeStruct((N_EDGES, EDGE_DIM_OUT), jnp.float32),
    )(edge_emb, g, we_t, b2d)


def kernel(edge_index, edge_emb, node_emb, W, b):
    ei = edge_index.astype(jnp.int32)
    src1 = ei[0]
    dst1 = ei[1]

    we_t = W[:, :EDGE_DIM].T
    ws_t = W[:, EDGE_DIM:EDGE_DIM + NODE_DIM].T
    wd_t = W[:, EDGE_DIM + NODE_DIM:].T

    ps, pd = _node_proj(node_emb, ws_t, wd_t)
    g = _sc_gather_add(ps, pd, src1, dst1)
    g2 = g.reshape(N_EDGES, EDGE_DIM_OUT)
    return _finish(edge_emb, g2, we_t, b.reshape(1, EDGE_DIM_OUT))
